# Initial kernel scaffold; baseline (speedup 1.0000x reference)
import jax
import jax.numpy as jnp
from jax.experimental import pallas as pl


def kernel(user, item, genres, writers, directors, year, emb_table, fc_table, bias, W1, b1, W2, b2, W3, b3):
    raise NotImplementedError("write your pallas kernel here")


# speedup vs baseline: 2.0587x; 2.0587x over previous
import functools

import jax
import jax.numpy as jnp
from jax import lax
from jax.experimental import pallas as pl
from jax.experimental.pallas import tpu as pltpu
from jax.experimental.pallas import tpu_sc as plsc

V = 900200
PAD = 600000
D = 16
B = 16384
NSLOT = 96
MLP_IN = 96
H1, H2 = 256, 128

SEGS = ((0, 1), (1, 2), (2, 22), (22, 72), (72, 92), (92, 93))

NC = 2
NS = 16
NW = NC * NS
PER_W = B // NW
CHUNK = 16
NCHUNK = PER_W // CHUNK
CIDX = CHUNK * NSLOT
GSUB = 128
NGATHER = CIDX // GSUB


def _sc_gather_pool(emb_table, fc_flat, idx_flat):
    mesh = plsc.VectorSubcoreMesh(core_axis_name="c", subcore_axis_name="s")

    @functools.partial(
        pl.kernel,
        mesh=mesh,
        compiler_params=pltpu.CompilerParams(use_tc_tiling_on_sc=False),
        out_type=(
            jax.ShapeDtypeStruct((B * NSLOT,), jnp.float32),
            jax.ShapeDtypeStruct((B * NSLOT,), jnp.float32),
        ),
        scratch_types=[
            pltpu.VMEM((CIDX,), jnp.int32),
            pltpu.VMEM((CIDX, D), jnp.float32),
            pltpu.VMEM((CIDX,), jnp.float32),
            pltpu.VMEM((CIDX,), jnp.float32),
            pltpu.SemaphoreType.DMA,
            pltpu.SemaphoreType.DMA,
        ],
    )
    def k(emb_hbm, fc_hbm, idx_hbm, sums_hbm, fcraw_hbm,
          idx_v, rows_v, fc_v, out_v, sem_e, sem_f):
        wid = lax.axis_index("s") * NC + lax.axis_index("c")
        wbase = wid * PER_W * NSLOT

        def chunk_body(c, carry):
            base = wbase + c * CIDX
            pltpu.sync_copy(idx_hbm.at[pl.ds(base, CIDX)], idx_v)
            cps = []
            for j in range(NGATHER):
                sl = pl.ds(j * GSUB, GSUB)
                cps.append(pltpu.async_copy(
                    emb_hbm.at[idx_v.at[sl]], rows_v.at[sl], sem_e))
                cps.append(pltpu.async_copy(
                    fc_hbm.at[idx_v.at[sl]], fc_v.at[sl], sem_f))
            for cp in cps:
                cp.wait()

            def s_body(s, carry2):
                rb = s * NSLOT
                for f, (lo, hi) in enumerate(SEGS):
                    acc = rows_v[rb + lo]
                    for r in range(lo + 1, hi):
                        acc = acc + rows_v[rb + r]
                    out_v[pl.ds(rb + f * D, D)] = acc
                return carry2

            lax.fori_loop(0, CHUNK, s_body, 0)
            pltpu.sync_copy(out_v, sums_hbm.at[pl.ds(base, CIDX)])
            pltpu.sync_copy(fc_v, fcraw_hbm.at[pl.ds(base, CIDX)])
            return carry

        lax.fori_loop(0, NCHUNK, chunk_body, 0)

    return k(emb_table, fc_flat, idx_flat)


def _tc_body(idx_ref, sums_ref, fc_ref, W1_ref, b1_ref, W2_ref, b2_ref,
             W3_ref, b3_ref, bias_ref, out_ref):
    idx = idx_ref[...]
    mask = (idx != PAD).astype(jnp.float32)
    col = lax.broadcasted_iota(jnp.int32, idx.shape, 1)

    def inv_count(lo, hi):
        seg = jnp.logical_and(col >= lo, col < hi)
        cnt = jnp.sum(jnp.where(seg, mask, 0.0), axis=1, keepdims=True)
        return seg, 1.0 / (cnt + 1e-08)

    seg_g, inv_g = inv_count(2, 22)
    seg_w, inv_w = inv_count(22, 72)
    seg_d, inv_d = inv_count(72, 92)

    slot_scale = jnp.where(seg_g, inv_g,
                           jnp.where(seg_w, inv_w,
                                     jnp.where(seg_d, inv_d, 1.0)))
    fm1 = bias_ref[0, 0] + jnp.sum(fc_ref[...] * slot_scale, axis=1,
                                   keepdims=True)

    fcol = col // D
    esc = jnp.where(fcol == 2, inv_g,
                    jnp.where(fcol == 3, inv_w,
                              jnp.where(fcol == 4, inv_d, 1.0)))
    ex = sums_ref[...] * esc

    s1 = jnp.sum(ex, axis=1, keepdims=True)
    s2 = jnp.sum(ex * ex, axis=1, keepdims=True)
    fm2 = 0.5 * (s1 * s1 - s2)

    h = jnp.maximum(
        jnp.dot(ex, W1_ref[...], preferred_element_type=jnp.float32)
        + b1_ref[...], 0.0)
    h = jnp.maximum(
        jnp.dot(h, W2_ref[...], preferred_element_type=jnp.float32)
        + b2_ref[...], 0.0)
    mlp = (jnp.dot(h, W3_ref[...], preferred_element_type=jnp.float32)
           + b3_ref[0, 0])

    out_ref[...] = jax.nn.sigmoid(fm1 + fm2 + mlp)


def kernel(user, item, genres, writers, directors, year,
           emb_table, fc_table, bias, W1, b1, W2, b2, W3, b3):
    i32 = jnp.int32
    idx_all = jnp.concatenate([
        user[:, None].astype(i32), item[:, None].astype(i32),
        genres.astype(i32), writers.astype(i32), directors.astype(i32),
        year[:, None].astype(i32),
        jnp.full((B, 3), PAD, dtype=i32),
    ], axis=1)

    sums_flat, fc_flat = _sc_gather_pool(
        emb_table, fc_table[:, 0], idx_all.reshape(-1))
    sums = sums_flat.reshape(B, NSLOT)
    fcraw = fc_flat.reshape(B, NSLOT)

    bm = 2048
    grid = (B // bm,)
    full = lambda i: (0, 0)
    y = pl.pallas_call(
        _tc_body,
        grid=grid,
        in_specs=[
            pl.BlockSpec((bm, NSLOT), lambda i: (i, 0)),
            pl.BlockSpec((bm, NSLOT), lambda i: (i, 0)),
            pl.BlockSpec((bm, NSLOT), lambda i: (i, 0)),
            pl.BlockSpec((MLP_IN, H1), full),
            pl.BlockSpec((1, H1), full),
            pl.BlockSpec((H1, H2), full),
            pl.BlockSpec((1, H2), full),
            pl.BlockSpec((H2, 1), full),
            pl.BlockSpec((1, 1), full),
            pl.BlockSpec((1, 1), full),
        ],
        out_specs=pl.BlockSpec((bm, 1), lambda i: (i, 0)),
        out_shape=jax.ShapeDtypeStruct((B, 1), jnp.float32),
    )(idx_all, sums, fcraw, W1, b1.reshape(1, H1), W2, b2.reshape(1, H2),
      W3, b3.reshape(1, 1), bias.reshape(1, 1))
    return y[:, 0]
